# Initial kernel scaffold; baseline (speedup 1.0000x reference)
import jax
import jax.numpy as jnp
from jax.experimental import pallas as pl


def kernel(nte, ete, neW1, neb1, neW2, neb2, eeW1, eeb1, eeW2, eeb2, ceW1, ceb1, ceW2, ceb2, cnW1, cnb1, cnW2, cnb2, ndW1, ndb1, ndW2, ndb2, ndW3, ndb3, edW1, edb1, edW2, edb2, edW3, edb3, x_node_types, x_edge_types, edge_index, steps):
    raise NotImplementedError("write your pallas kernel here")


# speedup vs baseline: 2.8003x; 2.8003x over previous
import functools

import jax
import jax.numpy as jnp
from jax import lax
from jax.experimental import pallas as pl
from jax.experimental.pallas import tpu as pltpu
from jax.experimental.pallas import tpu_sc as plsc

F = 16
NCORE = 2
NSUB = 16
NW = NCORE * NSUB
CH = 128
KJ = 8
GRP = KJ * CH

N = 50000
E = 800000
G = -(-(E // NW) // GRP)
E_PAD = NW * G * GRP
N_TAB = 50176
PER_SUB = N_TAB // NSUB
ZB = 784
DUMMY = N

BN = 1024
BE = 4096

_f32 = jnp.float32
_i32 = jnp.int32


def _relu(x):
    return jnp.maximum(x, 0.0)


def _onehot(tf_ref, rows):
    return (tf_ref[...].astype(_i32)
            == lax.broadcasted_iota(_i32, (rows, 8), 1)).astype(_f32)


def _tc_init_nodes(ntf, T0):
    def body(ntf_ref, T0_ref, a_ref, c_ref, u_ref):
        acu = jnp.dot(_onehot(ntf_ref, BN), T0_ref[...],
                      preferred_element_type=_f32)
        a_ref[...] = acu[:, 0:16]
        c_ref[...] = acu[:, 16:32]
        u_ref[...] = acu[:, 32:48]

    o = jax.ShapeDtypeStruct((N_TAB, F), _f32)
    return pl.pallas_call(
        body,
        grid=(N_TAB // BN,),
        in_specs=[pl.BlockSpec((BN, 1), lambda i: (i, 0)),
                  pl.BlockSpec((8, 48), lambda i: (0, 0))],
        out_specs=[pl.BlockSpec((BN, F), lambda i: (i, 0))] * 3,
        out_shape=[o, o, o],
    )(ntf, T0)


def _tc_init_edges(etf, Tb0):
    def body(etf_ref, Tb0_ref, eb_ref):
        eb_ref[...] = jnp.dot(_onehot(etf_ref, BE), Tb0_ref[...],
                              preferred_element_type=_f32)

    return pl.pallas_call(
        body,
        grid=(E_PAD // BE,),
        in_specs=[pl.BlockSpec((BE, 1), lambda i: (i, 0)),
                  pl.BlockSpec((8, F), lambda i: (0, 0))],
        out_specs=pl.BlockSpec((BE, F), lambda i: (i, 0)),
        out_shape=jax.ShapeDtypeStruct((E_PAD, F), _f32),
    )(etf, Tb0)


def _tc_edge(er1, etf, ceW2, ceb2, M2, Wb2, Tb):
    def body(er1_ref, etf_ref, W2_ref, b2_ref, M2_ref, Wb2_ref, Tb_ref,
             er_ref, w_ref, ebn_ref):
        er = _relu(jnp.dot(er1_ref[...], W2_ref[...],
                           preferred_element_type=_f32) + b2_ref[...])
        er_ref[...] = er
        w_ref[...] = jnp.dot(er, M2_ref[...], preferred_element_type=_f32)
        ebn_ref[...] = (jnp.dot(_onehot(etf_ref, BE), Tb_ref[...],
                                preferred_element_type=_f32)
                        + jnp.dot(er, Wb2_ref[...],
                                  preferred_element_type=_f32))

    full = lambda shape: pl.BlockSpec(shape, lambda i: (0, 0))
    o = jax.ShapeDtypeStruct((E_PAD, F), _f32)
    return pl.pallas_call(
        body,
        grid=(E_PAD // BE,),
        in_specs=[pl.BlockSpec((BE, F), lambda i: (i, 0)),
                  pl.BlockSpec((BE, 1), lambda i: (i, 0)),
                  full((F, F)), full((1, F)), full((F, F)), full((F, F)),
                  full((8, F))],
        out_specs=[pl.BlockSpec((BE, F), lambda i: (i, 0))] * 3,
        out_shape=[o, o, o],
    )(er1, etf, ceW2, ceb2, M2, Wb2, Tb)


def _tc_node(S0, S1, ntf, cnb1, cnW2, cnb2, TN, W3):
    def body(s0_ref, s1_ref, ntf_ref, b1_ref, W2_ref, b2_ref, TN_ref, W3_ref,
             xn_ref, a_ref, c_ref, u_ref):
        xn1 = _relu(s0_ref[...] + s1_ref[...] + b1_ref[...])
        xn = _relu(jnp.dot(xn1, W2_ref[...], preferred_element_type=_f32)
                   + b2_ref[...])
        xn_ref[...] = xn
        acu = (jnp.dot(_onehot(ntf_ref, BN), TN_ref[...],
                       preferred_element_type=_f32)
               + jnp.dot(xn, W3_ref[...], preferred_element_type=_f32))
        a_ref[...] = acu[:, 0:16]
        c_ref[...] = acu[:, 16:32]
        u_ref[...] = acu[:, 32:48]

    full = lambda shape: pl.BlockSpec(shape, lambda i: (0, 0))
    o = jax.ShapeDtypeStruct((N_TAB, F), _f32)
    return pl.pallas_call(
        body,
        grid=(N_TAB // BN,),
        in_specs=[pl.BlockSpec((BN, F), lambda i: (i, 0)),
                  pl.BlockSpec((BN, F), lambda i: (i, 0)),
                  pl.BlockSpec((BN, 1), lambda i: (i, 0)),
                  full((1, F)), full((F, F)), full((1, F)),
                  full((8, 48)), full((F, 48))],
        out_specs=[pl.BlockSpec((BN, F), lambda i: (i, 0))] * 4,
        out_shape=[o, o, o, o],
    )(S0, S1, ntf, cnb1, cnW2, cnb2, TN, W3)


def _tc_dec(x, W1, b1, W2, b2, W3p, b3p, rows, block):
    def body(x_ref, W1_ref, b1_ref, W2_ref, b2_ref, W3_ref, b3_ref, o_ref):
        h = _relu(jnp.dot(x_ref[...], W1_ref[...],
                          preferred_element_type=_f32) + b1_ref[...])
        h = _relu(jnp.dot(h, W2_ref[...], preferred_element_type=_f32)
                  + b2_ref[...])
        o_ref[...] = jnp.dot(h, W3_ref[...],
                             preferred_element_type=_f32) + b3_ref[...]

    full = lambda shape: pl.BlockSpec(shape, lambda i: (0, 0))
    return pl.pallas_call(
        body,
        grid=(rows // block,),
        in_specs=[pl.BlockSpec((block, F), lambda i: (i, 0)),
                  full((F, F)), full((1, F)), full((F, F)), full((1, F)),
                  full((F, 8)), full((1, 8))],
        out_specs=pl.BlockSpec((block, 8), lambda i: (i, 0)),
        out_shape=jax.ShapeDtypeStruct((rows, 8), _f32),
    )(x, W1, b1, W2, b2, W3p, b3p)


@functools.lru_cache(maxsize=None)
def _sc_gather_kernel():
    mesh = plsc.VectorSubcoreMesh(core_axis_name="c", subcore_axis_name="s")
    return functools.partial(
        pl.kernel, mesh=mesh,
        compiler_params=pltpu.CompilerParams(use_tc_tiling_on_sc=False),
        out_type=jax.ShapeDtypeStruct((E_PAD, F), _f32),
        scratch_types=[
            pltpu.VMEM((KJ, CH), _i32),
            pltpu.VMEM((KJ, CH), _i32),
            pltpu.VMEM((GRP, F), _f32),
            pltpu.VMEM((GRP, F), _f32),
            pltpu.VMEM((GRP, F), _f32),
            pltpu.VMEM((GRP, F), _f32),
            pltpu.SemaphoreType.DMA,
            pltpu.SemaphoreType.DMA,
        ])(_sc_gather_body)


def _sc_gather_body(a_hbm, c_hbm, eb_hbm, src2_hbm, dst2_hbm, out_hbm,
                    idxs, idxd, rowsA, rowsC, ebv, outv, semA, semC):
    wid = lax.axis_index("s") * NCORE + lax.axis_index("c")
    base_g = wid * G

    def grp_body(g, _):
        row128 = (base_g + g) * KJ
        off = (base_g + g) * GRP
        pltpu.sync_copy(src2_hbm.at[pl.ds(row128, KJ), :], idxs)
        pltpu.sync_copy(dst2_hbm.at[pl.ds(row128, KJ), :], idxd)
        pltpu.sync_copy(eb_hbm.at[pl.ds(off, GRP), :], ebv)
        cps = []
        for j in range(KJ):
            cps.append(pltpu.async_copy(
                a_hbm.at[idxs.at[j]], rowsA.at[pl.ds(j * CH, CH), :], semA))
            cps.append(pltpu.async_copy(
                c_hbm.at[idxd.at[j]], rowsC.at[pl.ds(j * CH, CH), :], semC))
        for cp in cps:
            cp.wait()

        def row_body(i, _):
            outv[i] = jnp.maximum(rowsA[i] + rowsC[i] + ebv[i], 0.0)
            return 0

        lax.fori_loop(0, GRP, row_body, 0)
        pltpu.sync_copy(outv, out_hbm.at[pl.ds(off, GRP), :])
        return 0

    lax.fori_loop(0, G, grp_body, 0)


@functools.lru_cache(maxsize=None)
def _sc_scatter_kernel():
    mesh = plsc.VectorSubcoreMesh(core_axis_name="c", subcore_axis_name="s")
    return functools.partial(
        pl.kernel, mesh=mesh,
        compiler_params=pltpu.CompilerParams(use_tc_tiling_on_sc=False),
        out_type=jax.ShapeDtypeStruct((NCORE, N_TAB, F), _f32),
        scratch_types=[
            pltpu.VMEM((KJ, CH), _i32),
            pltpu.VMEM((KJ, CH), _i32),
            pltpu.VMEM((GRP, F), _f32),
            pltpu.VMEM((GRP, F), _f32),
            pltpu.VMEM((GRP, F), _f32),
            pltpu.VMEM((ZB, F), _f32),
            pltpu.VMEM_SHARED((N_TAB, F), _f32),
            pltpu.SemaphoreType.DMA,
        ])(_sc_scatter_body)


def _sc_scatter_body(u_hbm, w_hbm, src2_hbm, dst2_hbm, out_hbm,
                     idxs, idxd, rowsU, wv, valv, zbuf, acc, semU):
    cid = lax.axis_index("c")
    sid = lax.axis_index("s")
    wid = sid * NCORE + cid
    base_g = wid * G

    def zb_body(i, _):
        zbuf[i] = jnp.zeros((F,), _f32)
        return 0

    lax.fori_loop(0, ZB, zb_body, 0)
    for r in range(PER_SUB // ZB):
        pltpu.sync_copy(zbuf, acc.at[pl.ds(sid * PER_SUB + r * ZB, ZB), :])
    plsc.subcore_barrier()

    def grp_body(g, _):
        row128 = (base_g + g) * KJ
        off = (base_g + g) * GRP
        pltpu.sync_copy(src2_hbm.at[pl.ds(row128, KJ), :], idxs)
        pltpu.sync_copy(dst2_hbm.at[pl.ds(row128, KJ), :], idxd)
        pltpu.sync_copy(w_hbm.at[pl.ds(off, GRP), :], wv)
        cps = []
        for j in range(KJ):
            cps.append(pltpu.async_copy(
                u_hbm.at[idxs.at[j]], rowsU.at[pl.ds(j * CH, CH), :], semU))
        for cp in cps:
            cp.wait()

        def row_body(i, _):
            valv[i] = rowsU[i] + wv[i]
            return 0

        lax.fori_loop(0, GRP, row_body, 0)
        for j in range(KJ):
            pltpu.sync_copy(valv.at[pl.ds(j * CH, CH), :],
                            acc.at[idxd.at[j]], add=True)
        return 0

    lax.fori_loop(0, G, grp_body, 0)
    plsc.subcore_barrier()

    for r in range(PER_SUB // ZB):
        pltpu.sync_copy(acc.at[pl.ds(sid * PER_SUB + r * ZB, ZB), :], zbuf)
        pltpu.sync_copy(zbuf,
                        out_hbm.at[cid, pl.ds(sid * PER_SUB + r * ZB, ZB), :])


def kernel(nte, ete, neW1, neb1, neW2, neb2, eeW1, eeb1, eeW2, eeb2,
           ceW1, ceb1, ceW2, ceb2, cnW1, cnb1, cnW2, cnb2,
           ndW1, ndb1, ndW2, ndb2, ndW3, ndb3,
           edW1, edb1, edW2, edb2, edW3, edb3,
           x_node_types, x_edge_types, edge_index, steps):
    relu = _relu

    def mlp2(x, W1, b1, W2, b2):
        return relu(relu(x @ W1 + b1) @ W2 + b2)

    ntab = mlp2(nte, neW1, neb1, neW2, neb2)
    etab = mlp2(ete, eeW1, eeb1, eeW2, eeb2)
    Wa1, Wa2 = ceW1[0:16], ceW1[16:32]
    Wb1, Wb2 = ceW1[32:48], ceW1[48:64]
    Wc1, Wc2 = ceW1[64:80], ceW1[80:96]
    Wu1, Wu2 = cnW1[0:16], cnW1[16:32]
    M2 = cnW1[32:48]

    def pad8(t):
        return jnp.pad(t, ((0, 8 - t.shape[0]), (0, 0)))

    TN = pad8(jnp.concatenate([ntab @ Wa1, ntab @ Wc1, ntab @ Wu1], axis=1))
    T0 = pad8(jnp.concatenate([ntab @ (Wa1 + Wa2), ntab @ (Wc1 + Wc2),
                               ntab @ (Wu1 + Wu2)], axis=1))
    W3 = jnp.concatenate([Wa2, Wc2, Wu2], axis=1)
    Tb = pad8(etab @ Wb1 + ceb1)
    Tb0 = pad8(etab @ (Wb1 + Wb2) + ceb1)

    row = lambda b: b.reshape(1, F)
    ceb2r, cnb1r, cnb2r = row(ceb2), row(cnb1), row(cnb2)
    ndb1r, ndb2r, edb1r, edb2r = row(ndb1), row(ndb2), row(edb1), row(edb2)
    padW3 = lambda W: jnp.pad(W, ((0, 0), (0, 8 - W.shape[1])))
    ndW3p, edW3p = padW3(ndW3), padW3(edW3)
    ndb3p = jnp.pad(ndb3, (0, 8 - ndb3.shape[0])).reshape(1, 8)
    edb3p = jnp.pad(edb3, (0, 8 - edb3.shape[0])).reshape(1, 8)

    src = edge_index[0].astype(_i32)
    dst = edge_index[1].astype(_i32)
    padE = E_PAD - E
    src2 = jnp.concatenate([src, jnp.full((padE,), DUMMY, _i32)]
                           ).reshape(E_PAD // CH, CH)
    dst2 = jnp.concatenate([dst, jnp.full((padE,), DUMMY, _i32)]
                           ).reshape(E_PAD // CH, CH)
    ntf = jnp.pad(x_node_types.astype(_f32), (0, N_TAB - N)
                  ).reshape(N_TAB, 1)
    etf = jnp.pad(x_edge_types.astype(_f32), (0, padE)).reshape(E_PAD, 1)

    a0, c0, u0 = _tc_init_nodes(ntf, T0)
    eb0 = _tc_init_edges(etf, Tb0)
    er0 = jnp.zeros((E_PAD, F), _f32)
    xn0 = jnp.zeros((N_TAB, F), _f32)

    def body(t, carry):
        a, c, u, eb, er, xn = carry
        er1 = _sc_gather_kernel()(a, c, eb, src2, dst2)
        er, w, ebn = _tc_edge(er1, etf, ceW2, ceb2r, M2, Wb2, Tb)
        S2 = _sc_scatter_kernel()(u, w, src2, dst2)
        xn, a, c, u = _tc_node(S2[0], S2[1], ntf, cnb1r, cnW2, cnb2r, TN, W3)
        return (a, c, u, ebn, er, xn)

    a, c, u, eb, er, xn = lax.fori_loop(
        0, steps, body, (a0, c0, u0, eb0, er0, xn0))

    pn = _tc_dec(xn, ndW1, ndb1r, ndW2, ndb2r, ndW3p, ndb3p, N_TAB, BN)
    pe = _tc_dec(er, edW1, edb1r, edW2, edb2r, edW3p, edb3p, E_PAD, BE)
    return (pn[:N, :3], pe[:E, :3])
